# baseline (speedup 1.0000x reference)
import functools

import jax
import jax.numpy as jnp
from jax.experimental import pallas as pl

_DIA = 20
_NDLG = 400
_T = _DIA * _NDLG
_D = 128
_INV_DEG = 1.0 / 22.0


def _body(qs_ref, l_ref, a_ref, v_ref, st_ref,
          fw_ref, fb_ref, w1_ref, b1_ref, w2_ref, b2_ref,
          w3_ref, b3_ref, w4_ref, b4_ref, out_ref, *, rows, groups):
    f32 = jnp.float32
    dot = functools.partial(jnp.dot, preferred_element_type=f32,
                            precision=jax.lax.Precision.DEFAULT)
    st = st_ref[...]
    qs = qs_ref[...]
    cond = qs[:, 0:1] >= qs[:, 1:2]
    f_l = l_ref[...] + jnp.where(cond, st[0:1, :], st[1:2, :])
    f_a = a_ref[...]
    f_v = v_ref[...]
    fw = fw_ref[...]
    fb = fb_ref[...]
    xs = [dot(f_l, fw) + fb, dot(f_a, fw) + fb, dot(f_v, fw) + fb]
    for w_ref, b_ref in ((w1_ref, b1_ref), (w2_ref, b2_ref),
                         (w3_ref, b3_ref), (w4_ref, b4_ref)):
        w = w_ref[...] * _INV_DEG
        b = b_ref[...]
        others = (xs[1] + xs[2], xs[0] + xs[2], xs[0] + xs[1])
        nxt = []
        for x, oth in zip(xs, others):
            bs = x.reshape(groups, _DIA, _D).sum(axis=1)
            blocksum = jnp.broadcast_to(
                bs[:, None, :], (groups, _DIA, _D)).reshape(rows, _D)
            pre = blocksum + oth
            nxt.append(jnp.maximum(dot(pre, w) + b, 0.0))
        xs = nxt
    for j, part in enumerate((f_l, xs[0], f_a, xs[1], f_v, xs[2])):
        out_ref[:, j * _D:(j + 1) * _D] = part


def kernel(a, v, l, dia_len, qmask, epoch, speaker_table, fc1_w, fc1_b,
           W1, b1, W2, b2, W3, b3, W4, b4):
    del dia_len, epoch
    grid = 4
    rows = _T // grid
    groups = rows // _DIA
    qs = qmask.transpose(1, 0, 2).reshape(_T, 2)
    row_spec = pl.BlockSpec((rows, _D), lambda i: (i, 0))
    q_spec = pl.BlockSpec((rows, 2), lambda i: (i, 0))

    def full(shape):
        return pl.BlockSpec(shape, lambda i: (0,) * len(shape))

    mats = [speaker_table, fc1_w, fc1_b.reshape(1, _D)]
    for W, b in ((W1, b1), (W2, b2), (W3, b3), (W4, b4)):
        mats += [W, b.reshape(1, _D)]
    in_specs = ([q_spec, row_spec, row_spec, row_spec]
                + [full(m.shape) for m in mats])
    return pl.pallas_call(
        functools.partial(_body, rows=rows, groups=groups),
        grid=(grid,),
        in_specs=in_specs,
        out_specs=pl.BlockSpec((rows, 6 * _D), lambda i: (i, 0)),
        out_shape=jax.ShapeDtypeStruct((_T, 6 * _D), jnp.float32),
    )(qs, l, a, v, *mats)


# speedup vs baseline: 1.0048x; 1.0048x over previous
import functools

import jax
import jax.numpy as jnp
from jax.experimental import pallas as pl

_DIA = 20
_NDLG = 400
_T = _DIA * _NDLG
_D = 128
_INV_DEG = 1.0 / 22.0


def _body(qs_ref, l_ref, a_ref, v_ref, st_ref,
          fw_ref, fb_ref, w1_ref, b1_ref, w2_ref, b2_ref,
          w3_ref, b3_ref, w4_ref, b4_ref, out_ref, *, rows, groups):
    f32 = jnp.float32
    dot = functools.partial(jnp.dot, preferred_element_type=f32,
                            precision=jax.lax.Precision.DEFAULT)
    st = st_ref[...]
    qs = qs_ref[...]
    cond = qs[:, 0:1] >= qs[:, 1:2]
    f_l = l_ref[...] + jnp.where(cond, st[0:1, :], st[1:2, :])
    f_a = a_ref[...]
    f_v = v_ref[...]
    fw = fw_ref[...]
    fb = fb_ref[...]
    xs = [dot(f_l, fw) + fb, dot(f_a, fw) + fb, dot(f_v, fw) + fb]
    for w_ref, b_ref in ((w1_ref, b1_ref), (w2_ref, b2_ref),
                         (w3_ref, b3_ref), (w4_ref, b4_ref)):
        w = w_ref[...] * _INV_DEG
        b = b_ref[...]
        others = (xs[1] + xs[2], xs[0] + xs[2], xs[0] + xs[1])
        nxt = []
        for x, oth in zip(xs, others):
            bs = x.reshape(groups, _DIA, _D).sum(axis=1)
            blocksum = jnp.broadcast_to(
                bs[:, None, :], (groups, _DIA, _D)).reshape(rows, _D)
            pre = blocksum + oth
            nxt.append(jnp.maximum(dot(pre, w) + b, 0.0))
        xs = nxt
    for j, part in enumerate((f_l, xs[0], f_a, xs[1], f_v, xs[2])):
        out_ref[:, j * _D:(j + 1) * _D] = part


def kernel(a, v, l, dia_len, qmask, epoch, speaker_table, fc1_w, fc1_b,
           W1, b1, W2, b2, W3, b3, W4, b4):
    del dia_len, epoch
    grid = 5
    rows = _T // grid
    groups = rows // _DIA
    qs = qmask.transpose(1, 0, 2).reshape(_T, 2)
    row_spec = pl.BlockSpec((rows, _D), lambda i: (i, 0))
    q_spec = pl.BlockSpec((rows, 2), lambda i: (i, 0))

    def full(shape):
        return pl.BlockSpec(shape, lambda i: (0,) * len(shape))

    mats = [speaker_table, fc1_w, fc1_b.reshape(1, _D)]
    for W, b in ((W1, b1), (W2, b2), (W3, b3), (W4, b4)):
        mats += [W, b.reshape(1, _D)]
    in_specs = ([q_spec, row_spec, row_spec, row_spec]
                + [full(m.shape) for m in mats])
    return pl.pallas_call(
        functools.partial(_body, rows=rows, groups=groups),
        grid=(grid,),
        in_specs=in_specs,
        out_specs=pl.BlockSpec((rows, 6 * _D), lambda i: (i, 0)),
        out_shape=jax.ShapeDtypeStruct((_T, 6 * _D), jnp.float32),
    )(qs, l, a, v, *mats)
